# Initial kernel scaffold; baseline (speedup 1.0000x reference)
import jax
import jax.numpy as jnp
from jax.experimental import pallas as pl


def kernel(strength, length, phrase, weight):
    raise NotImplementedError("write your pallas kernel here")


# speedup vs baseline: 1.9915x; 1.9915x over previous
import functools

import jax
import jax.numpy as jnp
from jax import lax
from jax.experimental import pallas as pl
from jax.experimental.pallas import tpu as pltpu
from jax.experimental.pallas import tpu_sc as plsc

B, S, D, V = 1024, 200, 128, 512
ROWS = B * S
NC, NS = 2, 16
NW = NC * NS
RPW = ROWS // NW
CHUNK = 128
NCHUNK = RPW // CHUNK
LANES = 16


def _posenc_tc():
    def body(o_ref):
        pos = lax.broadcasted_iota(jnp.int32, (S, D), 0).astype(jnp.float32)
        ch = lax.broadcasted_iota(jnp.int32, (S, D), 1)
        exp2i = ((ch // 2) * 2).astype(jnp.float32)
        inv_freq = jnp.exp(exp2i * (-jnp.log(10000.0) / D))
        ang = pos * inv_freq
        o_ref[...] = jnp.where(ch % 2 == 0, jnp.sin(ang), jnp.cos(ang))

    return pl.pallas_call(
        body, out_shape=jax.ShapeDtypeStruct((S, D), jnp.float32)
    )()


@functools.partial(
    pl.kernel,
    mesh=plsc.VectorSubcoreMesh(core_axis_name="c", subcore_axis_name="s"),
    out_type=jax.ShapeDtypeStruct((ROWS, D), jnp.float32),
    scratch_types=[
        pltpu.VMEM((CHUNK,), jnp.int32),
        pltpu.VMEM((CHUNK, D), jnp.float32),
        pltpu.VMEM((S, D), jnp.float32),
        pltpu.SemaphoreType.DMA,
    ],
)
def _sc_embed(idx_hbm, w_hbm, pos_hbm, out_hbm, idx_v, dest_v, pos_v, sem):
    wid = lax.axis_index("s") * NC + lax.axis_index("c")
    base = wid * RPW
    pltpu.sync_copy(pos_hbm, pos_v)

    def chunk_body(g, carry):
        row0 = base + g * CHUNK
        pltpu.sync_copy(idx_hbm.at[pl.ds(row0, CHUNK)], idx_v)
        pltpu.async_copy(w_hbm.at[idx_v], dest_v, sem).wait()
        poff = lax.rem(g * CHUNK, S)

        def row_body(r, carry2):
            pr = lax.rem(poff + r, S)
            for j in range(D // LANES):
                sl = pl.ds(j * LANES, LANES)
                plsc.addupdate(dest_v.at[r, sl], pos_v[pr, sl])
            return carry2

        lax.fori_loop(0, CHUNK, row_body, 0)
        pltpu.sync_copy(dest_v, out_hbm.at[pl.ds(row0, CHUNK)])
        return carry

    lax.fori_loop(0, NCHUNK, chunk_body, 0)


def kernel(strength, length, phrase, weight):
    del length, phrase
    pos = _posenc_tc()
    idx_flat = strength.reshape(ROWS).astype(jnp.int32)
    out = _sc_embed(idx_flat, weight.astype(jnp.float32), pos)
    return out.reshape(B, S, D)


# speedup vs baseline: 2.6808x; 1.3461x over previous
import functools

import jax
import jax.numpy as jnp
from jax import lax
from jax.experimental import pallas as pl
from jax.experimental.pallas import tpu as pltpu
from jax.experimental.pallas import tpu_sc as plsc

B, S, D, V = 1024, 200, 128, 512
ROWS = B * S
NC, NS = 2, 16
NW = NC * NS
RPW = ROWS // NW
CHUNK = 128
NCHUNK = RPW // CHUNK
LANES = 16


def _posenc_tc():
    def body(o_ref):
        pos = lax.broadcasted_iota(jnp.int32, (S, D), 0).astype(jnp.float32)
        ch = lax.broadcasted_iota(jnp.int32, (S, D), 1)
        exp2i = ((ch // 2) * 2).astype(jnp.float32)
        inv_freq = jnp.exp(exp2i * (-jnp.log(10000.0) / D))
        ang = pos * inv_freq
        o_ref[...] = jnp.where(ch % 2 == 0, jnp.sin(ang), jnp.cos(ang))

    return pl.pallas_call(
        body, out_shape=jax.ShapeDtypeStruct((S, D), jnp.float32)
    )()


@functools.partial(
    pl.kernel,
    mesh=plsc.VectorSubcoreMesh(core_axis_name="c", subcore_axis_name="s"),
    out_type=jax.ShapeDtypeStruct((ROWS, D), jnp.float32),
    scratch_types=[
        pltpu.VMEM((2, CHUNK), jnp.int32),
        pltpu.VMEM((2, CHUNK, D), jnp.float32),
        pltpu.VMEM((S, D), jnp.float32),
        pltpu.SemaphoreType.DMA,
        pltpu.SemaphoreType.DMA,
        pltpu.SemaphoreType.DMA,
        pltpu.SemaphoreType.DMA,
    ],
)
def _sc_embed(idx_hbm, w_hbm, pos_hbm, out_hbm, idx_v, dest_v, pos_v,
              sem_g0, sem_g1, sem_s0, sem_s1):
    wid = lax.axis_index("s") * NC + lax.axis_index("c")
    base = wid * RPW
    sem_g = (sem_g0, sem_g1)
    sem_s = (sem_s0, sem_s1)
    pltpu.sync_copy(pos_hbm, pos_v)

    def add_posenc(buf, g):
        poff = lax.rem(g * CHUNK, S)

        def row_body(r, carry2):
            pr = lax.rem(poff + r, S)
            for j in range(D // LANES):
                sl = pl.ds(j * LANES, LANES)
                plsc.addupdate(dest_v.at[buf, r, sl], pos_v[pr, sl])
            return carry2

        lax.fori_loop(0, CHUNK, row_body, 0)

    pltpu.sync_copy(idx_hbm.at[pl.ds(base, CHUNK)], idx_v.at[0])
    pltpu.async_copy(w_hbm.at[idx_v.at[0]], dest_v.at[0], sem_g[0])

    def pair_body(p, carry):
        for b in range(2):
            g = 2 * p + b
            row0 = base + g * CHUNK
            bn = 1 - b
            pltpu.make_async_copy(
                w_hbm.at[idx_v.at[b]], dest_v.at[b], sem_g[b]).wait()

            def start_next():
                gn = g + 1
                rown = base + gn * CHUNK

                def drain_prev_scatter():
                    pltpu.make_async_copy(
                        dest_v.at[bn],
                        out_hbm.at[pl.ds(row0 - CHUNK, CHUNK)],
                        sem_s[bn]).wait()

                if b == 0:
                    pl.when(p >= 1)(drain_prev_scatter)
                else:
                    drain_prev_scatter()
                pltpu.sync_copy(idx_hbm.at[pl.ds(rown, CHUNK)], idx_v.at[bn])
                pltpu.async_copy(w_hbm.at[idx_v.at[bn]], dest_v.at[bn],
                                 sem_g[bn])

            if b == 0:
                start_next()
            else:
                pl.when(p < NCHUNK // 2 - 1)(start_next)

            add_posenc(b, g)
            pltpu.async_copy(dest_v.at[b], out_hbm.at[pl.ds(row0, CHUNK)],
                             sem_s[b])
        return carry

    lax.fori_loop(0, NCHUNK // 2, pair_body, 0)

    for b in range(2):
        g = NCHUNK - 2 + b
        pltpu.make_async_copy(
            dest_v.at[b], out_hbm.at[pl.ds(base + g * CHUNK, CHUNK)],
            sem_s[b]).wait()


def kernel(strength, length, phrase, weight):
    del length, phrase
    pos = _posenc_tc()
    idx_flat = strength.reshape(ROWS).astype(jnp.int32)
    out = _sc_embed(idx_flat, weight.astype(jnp.float32), pos)
    return out.reshape(B, S, D)


# speedup vs baseline: 2.8502x; 1.0632x over previous
import functools

import jax
import jax.numpy as jnp
from jax import lax
from jax.experimental import pallas as pl
from jax.experimental.pallas import tpu as pltpu
from jax.experimental.pallas import tpu_sc as plsc

B, S, D, V = 1024, 200, 128, 512
ROWS = B * S
NC, NS = 2, 16
NW = NC * NS
RPW = ROWS // NW
CHUNK = 128
NCHUNK = RPW // CHUNK
LANES = 16


def _posenc_tc():
    def body(o_ref):
        pos = lax.broadcasted_iota(jnp.int32, (S, D), 0).astype(jnp.float32)
        ch = lax.broadcasted_iota(jnp.int32, (S, D), 1)
        exp2i = ((ch // 2) * 2).astype(jnp.float32)
        inv_freq = jnp.exp(exp2i * (-jnp.log(10000.0) / D))
        ang = pos * inv_freq
        o_ref[...] = jnp.where(ch % 2 == 0, jnp.sin(ang), jnp.cos(ang))

    return pl.pallas_call(
        body, out_shape=jax.ShapeDtypeStruct((S, D), jnp.float32)
    )()


BB = 128
NB = B // BB
UNITS = S * NB
UPW = UNITS // NW


@functools.partial(
    pl.kernel,
    mesh=plsc.VectorSubcoreMesh(core_axis_name="c", subcore_axis_name="s"),
    out_type=jax.ShapeDtypeStruct((B, S * D), jnp.float32),
    scratch_types=[
        pltpu.VMEM((2, BB), jnp.int32),
        pltpu.VMEM((2, BB, D), jnp.float32),
        pltpu.VMEM((S, D), jnp.float32),
        pltpu.SemaphoreType.DMA,
        pltpu.SemaphoreType.DMA,
        pltpu.SemaphoreType.DMA,
        pltpu.SemaphoreType.DMA,
    ],
)
def _sc_embed(idxt_hbm, w_hbm, pos_hbm, out_hbm, idx_v, dest_v, pos_v,
              sem_g0, sem_g1, sem_s0, sem_s1):
    wid = lax.axis_index("s") * NC + lax.axis_index("c")
    base_u = wid * UPW
    sem_g = (sem_g0, sem_g1)
    sem_s = (sem_s0, sem_s1)
    pltpu.sync_copy(pos_hbm, pos_v)

    def unit_coords(u):
        s = lax.div(u, NB)
        bq = lax.rem(u, NB) * BB
        return s, bq

    def out_block(u):
        s, bq = unit_coords(u)
        return out_hbm.at[pl.ds(bq, BB), pl.ds(s * D, D)]

    def start_gather(u, buf):
        s, bq = unit_coords(u)
        pltpu.sync_copy(idxt_hbm.at[pl.ds(s * B + bq, BB)], idx_v.at[buf])
        pltpu.async_copy(w_hbm.at[idx_v.at[buf]], dest_v.at[buf], sem_g[buf])

    def add_posenc(buf, u):
        s, _ = unit_coords(u)
        pvs = [pos_v[s, pl.ds(j * LANES, LANES)] for j in range(D // LANES)]

        def row_body(r, carry2):
            for j in range(D // LANES):
                plsc.addupdate(dest_v.at[buf, r, pl.ds(j * LANES, LANES)],
                               pvs[j])
            return carry2

        lax.fori_loop(0, BB, row_body, 0)

    start_gather(base_u, 0)

    def pair_body(p, carry):
        for b in range(2):
            u = base_u + 2 * p + b
            bn = 1 - b
            pltpu.make_async_copy(
                w_hbm.at[idx_v.at[b]], dest_v.at[b], sem_g[b]).wait()

            def start_next():
                def drain_prev_scatter():
                    pltpu.make_async_copy(
                        dest_v.at[bn], out_block(u - 1), sem_s[bn]).wait()

                if b == 0:
                    pl.when(p >= 1)(drain_prev_scatter)
                else:
                    drain_prev_scatter()
                start_gather(u + 1, bn)

            if b == 0:
                start_next()
            else:
                pl.when(p < UPW // 2 - 1)(start_next)

            add_posenc(b, u)
            pltpu.async_copy(dest_v.at[b], out_block(u), sem_s[b])
        return carry

    lax.fori_loop(0, UPW // 2, pair_body, 0)

    for b in range(2):
        u = base_u + UPW - 2 + b
        pltpu.make_async_copy(dest_v.at[b], out_block(u), sem_s[b]).wait()


def kernel(strength, length, phrase, weight):
    del length, phrase
    pos = _posenc_tc()
    idx_t = strength.astype(jnp.int32).T.reshape(S * B)
    out = _sc_embed(idx_t, weight.astype(jnp.float32), pos)
    return out.reshape(B, S, D)
